# baseline (speedup 1.0000x reference)
import functools

import jax
import jax.numpy as jnp
from jax import lax
from jax.experimental import pallas as pl
from jax.experimental.pallas import tpu as pltpu
from jax.experimental.pallas import tpu_sc as plsc

NC = 2
NS = 16
L = 16
NW = NC * NS
ALIGN = NW * 1024

F32 = jnp.float32
I32 = jnp.int32

_SC_PARAMS = pltpu.CompilerParams(needs_layout_passes=False)


def _mesh():
    return plsc.VectorSubcoreMesh(core_axis_name="c", subcore_axis_name="s",
                                  num_cores=NC, num_subcores=NS)


def _fill(buf, n, val):
    vv = jnp.full((L,), val, F32)
    un = 8 if n % (8 * L) == 0 else 1

    def body(i, _):
        for j in range(un):
            buf[pl.ds((i * un + j) * L, L)] = vv
        return 0

    lax.fori_loop(0, n // (un * L), body, 0)


def _zero_fill(buf, n):
    _fill(buf, n, 0.0)


def _sc_deg(dst_flat, n_pad):
    e_pad = dst_flat.shape[0]
    er = e_pad // NW
    n_sl = n_pad // NS

    @functools.partial(
        pl.kernel,
        out_type=jax.ShapeDtypeStruct((NC * n_pad,), F32),
        mesh=_mesh(),
        compiler_params=_SC_PARAMS,
        scratch_types=[
            pltpu.VMEM((er,), I32),
            pltpu.VMEM((er,), F32),
            pltpu.VMEM((n_sl,), F32),
            pltpu.VMEM_SHARED((n_pad,), F32),
        ],
    )
    def k(dst_hbm, deg_out, dst_v, ones_v, zrb_v, acc):
        cid = lax.axis_index("c")
        sid = lax.axis_index("s")
        wid = cid * NS + sid
        _zero_fill(zrb_v, n_sl)
        _fill(ones_v, er, 1.0)
        pltpu.sync_copy(zrb_v, acc.at[pl.ds(sid * n_sl, n_sl)])
        plsc.subcore_barrier()
        pltpu.sync_copy(dst_hbm.at[pl.ds(wid * er, er)], dst_v)
        pltpu.sync_copy(ones_v, acc.at[dst_v], add=True)
        plsc.subcore_barrier()
        pltpu.sync_copy(acc.at[pl.ds(sid * n_sl, n_sl)], zrb_v)
        pltpu.sync_copy(zrb_v, deg_out.at[pl.ds(cid * n_pad + sid * n_sl,
                                                n_sl)])

    return k(dst_flat)


def _tc_y_dis_u0(x, W1, W2, deg_n):
    n, d_in = x.shape
    blk = 400
    grid = n // blk

    def body(x_ref, w1_ref, w2_ref, d0_ref, d1_ref, dis_ref, u0_ref):
        w = w1_ref[...] @ w2_ref[...]
        y = x_ref[...] @ w
        d = d0_ref[:, 0] + d1_ref[:, 0] + jnp.float32(1.0)
        dis = lax.rsqrt(d)
        dis_ref[...] = dis[:, None]
        u0_ref[...] = (dis * y[:, 0])[:, None]

    d0, d1 = deg_n
    dis, u0 = pl.pallas_call(
        body,
        grid=(grid,),
        in_specs=[
            pl.BlockSpec((blk, d_in), lambda i: (i, 0)),
            pl.BlockSpec(W1.shape, lambda i: (0, 0)),
            pl.BlockSpec(W2.shape, lambda i: (0, 0)),
            pl.BlockSpec((blk, 1), lambda i: (i, 0)),
            pl.BlockSpec((blk, 1), lambda i: (i, 0)),
        ],
        out_specs=[pl.BlockSpec((blk, 1), lambda i: (i, 0)),
                   pl.BlockSpec((blk, 1), lambda i: (i, 0))],
        out_shape=(jax.ShapeDtypeStruct((n, 1), F32),
                   jax.ShapeDtypeStruct((n, 1), F32)),
    )(x, W1, W2, d0, d1)
    return dis.reshape(n), u0.reshape(n)


def _sc_hop(src_flat, dst_flat, u):
    n_pad = u.shape[0]
    e_pad = src_flat.shape[0]
    er = e_pad // NW
    nblk = 4
    cb = er // nblk
    n_sl = n_pad // NS

    @functools.partial(
        pl.kernel,
        out_type=jax.ShapeDtypeStruct((NC * n_pad,), F32),
        mesh=_mesh(),
        compiler_params=_SC_PARAMS,
        scratch_types=[
            pltpu.VMEM((n_pad,), F32),
            pltpu.VMEM((cb,), I32),
            pltpu.VMEM((cb,), I32), pltpu.VMEM((cb,), I32),
            pltpu.VMEM((cb,), F32), pltpu.VMEM((cb,), F32),
            pltpu.VMEM((n_sl,), F32),
            pltpu.VMEM_SHARED((n_pad,), F32),
            pltpu.SemaphoreType.DMA,
            pltpu.SemaphoreType.DMA,
            pltpu.SemaphoreType.DMA,
        ],
    )
    def k(src_hbm, dst_hbm, u_hbm, t_out, u_v, src_v, dst_v0, dst_v1,
          msg_v0, msg_v1, zrb_v, acc, usem, ssem0, ssem1):
        cid = lax.axis_index("c")
        sid = lax.axis_index("s")
        wid = cid * NS + sid
        ucp = pltpu.async_copy(u_hbm, u_v, usem)
        _zero_fill(zrb_v, n_sl)
        pltpu.sync_copy(zrb_v, acc.at[pl.ds(sid * n_sl, n_sl)])
        ucp.wait()
        plsc.subcore_barrier()
        dst_bufs = (dst_v0, dst_v1)
        msg_bufs = (msg_v0, msg_v1)
        sems = (ssem0, ssem1)
        scat = [None, None]
        for b in range(nblk):
            base = wid * er + b * cb
            dst_v = dst_bufs[b % 2]
            msg_v = msg_bufs[b % 2]
            if scat[b % 2] is not None:
                scat[b % 2].wait()
            pltpu.sync_copy(src_hbm.at[pl.ds(base, cb)], src_v)
            pltpu.sync_copy(dst_hbm.at[pl.ds(base, cb)], dst_v)

            def gath(i, _):
                for j in range(8):
                    s = pl.ds((i * 8 + j) * L, L)
                    msg_v[s] = plsc.load_gather(u_v, [src_v[s]])
                return 0

            lax.fori_loop(0, cb // (8 * L), gath, 0)
            scat[b % 2] = pltpu.async_copy(msg_v, acc.at[dst_v], sems[b % 2],
                                           add=True)
        scat[(nblk - 2) % 2].wait()
        scat[(nblk - 1) % 2].wait()
        plsc.subcore_barrier()
        pltpu.sync_copy(acc.at[pl.ds(sid * n_sl, n_sl)], zrb_v)
        pltpu.sync_copy(zrb_v, t_out.at[pl.ds(cid * n_pad + sid * n_sl,
                                              n_sl)])

    return k(src_flat, dst_flat, u)


def _tc_u_next(t_p, u, dis):
    n_pad = u.shape[0]
    r = n_pad // 128
    t3 = t_p.reshape(NC, r, 128)

    def body(t_ref, u_ref, dis_ref, un_ref):
        d = dis_ref[...]
        un_ref[...] = d * d * (t_ref[0] + t_ref[1] + u_ref[...])

    un = pl.pallas_call(
        body,
        out_shape=jax.ShapeDtypeStruct((r, 128), F32),
    )(t3, u.reshape(r, 128), dis.reshape(r, 128))
    return un.reshape(n_pad)


def _sc_pool(t_p, u, dis, batch_pad, g_pad):
    n_pad = u.shape[0]
    ch = n_pad // NW
    g_sl = g_pad // NS

    @functools.partial(
        pl.kernel,
        out_type=(jax.ShapeDtypeStruct((NC * g_pad,), F32),
                  jax.ShapeDtypeStruct((NC * g_pad,), F32)),
        mesh=_mesh(),
        compiler_params=_SC_PARAMS,
        scratch_types=[
            pltpu.VMEM((ch,), F32), pltpu.VMEM((ch,), F32),
            pltpu.VMEM((ch,), F32), pltpu.VMEM((ch,), F32),
            pltpu.VMEM((ch,), I32),
            pltpu.VMEM((g_pad,), F32), pltpu.VMEM((g_pad,), F32),
            pltpu.VMEM((g_sl,), F32), pltpu.VMEM((g_sl,), F32),
            pltpu.VMEM((g_sl,), F32),
            pltpu.VMEM_SHARED((NS * g_pad,), F32),
            pltpu.VMEM_SHARED((NS * g_pad,), F32),
        ],
    )
    def k(tp_hbm, u_hbm, dis_hbm, bat_hbm, pool_out, cnt_out, t0_v, t1_v,
          u_v, dis_v, bat_v, locp_v, locc_v, sump_v, sumc_v, tmp_v,
          stage_p, stage_c):
        cid = lax.axis_index("c")
        sid = lax.axis_index("s")
        wid = cid * NS + sid
        base = wid * ch
        _zero_fill(locp_v, g_pad)
        _zero_fill(locc_v, g_pad)
        pltpu.sync_copy(tp_hbm.at[pl.ds(base, ch)], t0_v)
        pltpu.sync_copy(tp_hbm.at[pl.ds(n_pad + base, ch)], t1_v)
        pltpu.sync_copy(u_hbm.at[pl.ds(base, ch)], u_v)
        pltpu.sync_copy(dis_hbm.at[pl.ds(base, ch)], dis_v)
        pltpu.sync_copy(bat_hbm.at[pl.ds(base, ch)], bat_v)
        ov = jnp.ones((L,), F32)

        def zbody(i, _):
            s = pl.ds(i * L, L)
            z = dis_v[s] * (t0_v[s] + t1_v[s] + u_v[s])
            idx = bat_v[s]
            plsc.addupdate_scatter(locp_v, [idx], z)
            plsc.addupdate_scatter(locc_v, [idx], ov)
            return 0

        lax.fori_loop(0, ch // L, zbody, 0)
        pltpu.sync_copy(locp_v, stage_p.at[pl.ds(sid * g_pad, g_pad)])
        pltpu.sync_copy(locc_v, stage_c.at[pl.ds(sid * g_pad, g_pad)])
        plsc.subcore_barrier()
        _zero_fill(sump_v, g_sl)
        _zero_fill(sumc_v, g_sl)
        for j in range(NS):
            pltpu.sync_copy(stage_p.at[pl.ds(j * g_pad + sid * g_sl, g_sl)],
                            tmp_v)
            for i in range(g_sl // L):
                s = pl.ds(i * L, L)
                sump_v[s] = sump_v[s] + tmp_v[s]
            pltpu.sync_copy(stage_c.at[pl.ds(j * g_pad + sid * g_sl, g_sl)],
                            tmp_v)
            for i in range(g_sl // L):
                s = pl.ds(i * L, L)
                sumc_v[s] = sumc_v[s] + tmp_v[s]
        pltpu.sync_copy(sump_v, pool_out.at[pl.ds(cid * g_pad + sid * g_sl,
                                                  g_sl)])
        pltpu.sync_copy(sumc_v, cnt_out.at[pl.ds(cid * g_pad + sid * g_sl,
                                                 g_sl)])

    return k(t_p, u, dis, batch_pad)


def _tc_final(pool_p, cnt_p, b1, W2, b2, g_pad):
    gr = g_pad // 128

    def body(pool_ref, cnt_ref, b1_ref, w2_ref, b2_ref, o_ref):
        c1s = jnp.sum(b1_ref[...] * w2_ref[...])
        pooled = pool_ref[0] + pool_ref[1]
        cnt = cnt_ref[0] + cnt_ref[1]
        o_ref[...] = pooled + cnt * c1s + b2_ref[0, 0]

    out = pl.pallas_call(
        body,
        out_shape=jax.ShapeDtypeStruct((gr, 128), F32),
    )(pool_p.reshape(NC, gr, 128), cnt_p.reshape(NC, gr, 128),
      b1.reshape(1, -1), W2.reshape(1, -1), b2.reshape(1, 1))
    return out.reshape(g_pad)


def kernel(x, edge_index, batch, W1, b1, W2, b2):
    n = x.shape[0]
    e = edge_index.shape[1]
    g = 512
    g_pad = 1024
    n_pad = ((n + 1 + ALIGN - 1) // ALIGN) * ALIGN
    e_pad = ((e + ALIGN - 1) // ALIGN) * ALIGN

    src_flat = jnp.concatenate([edge_index[0], jnp.zeros((e_pad - e,), I32)])
    dst_flat = jnp.concatenate([edge_index[1], jnp.full((e_pad - e,), n, I32)])
    batch_pad = jnp.concatenate([batch, jnp.full((n_pad - n,), g, I32)])

    deg_p = _sc_deg(dst_flat, n_pad)
    deg_n = (deg_p[:n].reshape(n, 1), deg_p[n_pad:n_pad + n].reshape(n, 1))
    dis_n, u0_n = _tc_y_dis_u0(x, W1, W2, deg_n)
    zpad = jnp.zeros((n_pad - n,), F32)
    dis = jnp.concatenate([dis_n, zpad])
    u0 = jnp.concatenate([u0_n, zpad])
    t1_p = _sc_hop(src_flat, dst_flat, u0)
    u1 = _tc_u_next(t1_p, u0, dis)
    t2_p = _sc_hop(src_flat, dst_flat, u1)
    pool_p, cnt_p = _sc_pool(t2_p, u1, dis, batch_pad, g_pad)
    out = _tc_final(pool_p, cnt_p, b1, W2, b2, g_pad)
    return out[:g].reshape(g, 1)


# speedup vs baseline: 1.2669x; 1.2669x over previous
import functools

import jax
import jax.numpy as jnp
from jax import lax
from jax.experimental import pallas as pl
from jax.experimental.pallas import tpu as pltpu
from jax.experimental.pallas import tpu_sc as plsc

NC = 2
NS = 16
L = 16
NW = NC * NS
ALIGN = NW * 1024

F32 = jnp.float32
I32 = jnp.int32

_SC_PARAMS = pltpu.CompilerParams(needs_layout_passes=False)


def _mesh():
    return plsc.VectorSubcoreMesh(core_axis_name="c", subcore_axis_name="s",
                                  num_cores=NC, num_subcores=NS)


def _fill(buf, n, val):
    vv = jnp.full((L,), val, F32)
    un = 8 if n % (8 * L) == 0 else 1

    def body(i, _):
        for j in range(un):
            buf[pl.ds((i * un + j) * L, L)] = vv
        return 0

    lax.fori_loop(0, n // (un * L), body, 0)


def _zero_fill(buf, n):
    _fill(buf, n, 0.0)


def _sc_deg(dst_flat, n_pad):
    e_pad = dst_flat.shape[0]
    er = e_pad // NW
    n_sl = n_pad // NS

    @functools.partial(
        pl.kernel,
        out_type=jax.ShapeDtypeStruct((NC * n_pad,), F32),
        mesh=_mesh(),
        compiler_params=_SC_PARAMS,
        scratch_types=[
            pltpu.VMEM((er,), I32),
            pltpu.VMEM((er,), F32),
            pltpu.VMEM((n_sl,), F32),
            pltpu.VMEM_SHARED((n_pad,), F32),
        ],
    )
    def k(dst_hbm, deg_out, dst_v, ones_v, zrb_v, acc):
        cid = lax.axis_index("c")
        sid = lax.axis_index("s")
        wid = cid * NS + sid
        _zero_fill(zrb_v, n_sl)
        _fill(ones_v, er, 1.0)
        pltpu.sync_copy(zrb_v, acc.at[pl.ds(sid * n_sl, n_sl)])
        plsc.subcore_barrier()
        pltpu.sync_copy(dst_hbm.at[pl.ds(wid * er, er)], dst_v)
        pltpu.sync_copy(ones_v, acc.at[dst_v], add=True)
        plsc.subcore_barrier()
        pltpu.sync_copy(acc.at[pl.ds(sid * n_sl, n_sl)], zrb_v)
        pltpu.sync_copy(zrb_v, deg_out.at[pl.ds(cid * n_pad + sid * n_sl,
                                                n_sl)])

    return k(dst_flat)


def _tc_matvec(x, W1, W2):
    n, d_in = x.shape
    blk = 400
    grid = n // blk

    def body(x_ref, w1_ref, w2_ref, y_ref):
        w = w1_ref[...] @ w2_ref[...]
        y_ref[...] = x_ref[...] @ w

    return pl.pallas_call(
        body,
        grid=(grid,),
        in_specs=[
            pl.BlockSpec((blk, d_in), lambda i: (i, 0)),
            pl.BlockSpec(W1.shape, lambda i: (0, 0)),
            pl.BlockSpec(W2.shape, lambda i: (0, 0)),
        ],
        out_specs=pl.BlockSpec((blk, 1), lambda i: (i, 0)),
        out_shape=jax.ShapeDtypeStruct((n, 1), F32),
    )(x, W1, W2)


def _tc_dis_u0(deg_p, y_pad):
    n_pad = y_pad.shape[0]
    r = n_pad // 128
    deg3 = deg_p.reshape(NC, r, 128)
    y2 = y_pad.reshape(r, 128)

    def body(deg_ref, y_ref, dis_ref, u0_ref):
        d = deg_ref[0] + deg_ref[1] + jnp.float32(1.0)
        dis = lax.rsqrt(d)
        dis_ref[...] = dis
        u0_ref[...] = dis * y_ref[...]

    dis, u0 = pl.pallas_call(
        body,
        out_shape=(jax.ShapeDtypeStruct((r, 128), F32),
                   jax.ShapeDtypeStruct((r, 128), F32)),
    )(deg3, y2)
    return dis.reshape(n_pad), u0.reshape(n_pad)


def _sc_hop(src_flat, dst_flat, u):
    n_pad = u.shape[0]
    e_pad = src_flat.shape[0]
    er = e_pad // NW
    nblk = 4
    cb = er // nblk
    n_sl = n_pad // NS

    @functools.partial(
        pl.kernel,
        out_type=jax.ShapeDtypeStruct((NC * n_pad,), F32),
        mesh=_mesh(),
        compiler_params=_SC_PARAMS,
        scratch_types=[
            pltpu.VMEM((n_pad,), F32),
            pltpu.VMEM((cb,), I32),
            pltpu.VMEM((cb,), I32), pltpu.VMEM((cb,), I32),
            pltpu.VMEM((cb,), F32), pltpu.VMEM((cb,), F32),
            pltpu.VMEM((n_sl,), F32),
            pltpu.VMEM_SHARED((n_pad,), F32),
            pltpu.SemaphoreType.DMA,
            pltpu.SemaphoreType.DMA,
            pltpu.SemaphoreType.DMA,
        ],
    )
    def k(src_hbm, dst_hbm, u_hbm, t_out, u_v, src_v, dst_v0, dst_v1,
          msg_v0, msg_v1, zrb_v, acc, usem, ssem0, ssem1):
        cid = lax.axis_index("c")
        sid = lax.axis_index("s")
        wid = cid * NS + sid
        ucp = pltpu.async_copy(u_hbm, u_v, usem)
        _zero_fill(zrb_v, n_sl)
        pltpu.sync_copy(zrb_v, acc.at[pl.ds(sid * n_sl, n_sl)])
        ucp.wait()
        plsc.subcore_barrier()
        dst_bufs = (dst_v0, dst_v1)
        msg_bufs = (msg_v0, msg_v1)
        sems = (ssem0, ssem1)
        scat = [None, None]
        for b in range(nblk):
            base = wid * er + b * cb
            dst_v = dst_bufs[b % 2]
            msg_v = msg_bufs[b % 2]
            if scat[b % 2] is not None:
                scat[b % 2].wait()
            pltpu.sync_copy(src_hbm.at[pl.ds(base, cb)], src_v)
            pltpu.sync_copy(dst_hbm.at[pl.ds(base, cb)], dst_v)

            def gath(i, _):
                for j in range(8):
                    s = pl.ds((i * 8 + j) * L, L)
                    msg_v[s] = plsc.load_gather(u_v, [src_v[s]])
                return 0

            lax.fori_loop(0, cb // (8 * L), gath, 0)
            scat[b % 2] = pltpu.async_copy(msg_v, acc.at[dst_v], sems[b % 2],
                                           add=True)
        scat[(nblk - 2) % 2].wait()
        scat[(nblk - 1) % 2].wait()
        plsc.subcore_barrier()
        pltpu.sync_copy(acc.at[pl.ds(sid * n_sl, n_sl)], zrb_v)
        pltpu.sync_copy(zrb_v, t_out.at[pl.ds(cid * n_pad + sid * n_sl,
                                              n_sl)])

    return k(src_flat, dst_flat, u)


def _tc_u_next(t_p, u, dis):
    n_pad = u.shape[0]
    r = n_pad // 128
    t3 = t_p.reshape(NC, r, 128)

    def body(t_ref, u_ref, dis_ref, un_ref):
        d = dis_ref[...]
        un_ref[...] = d * d * (t_ref[0] + t_ref[1] + u_ref[...])

    un = pl.pallas_call(
        body,
        out_shape=jax.ShapeDtypeStruct((r, 128), F32),
    )(t3, u.reshape(r, 128), dis.reshape(r, 128))
    return un.reshape(n_pad)


def _sc_pool(t_p, u, dis, batch_pad, g_pad):
    n_pad = u.shape[0]
    ch = n_pad // NW
    g_sl = g_pad // NS

    @functools.partial(
        pl.kernel,
        out_type=(jax.ShapeDtypeStruct((NC * g_pad,), F32),
                  jax.ShapeDtypeStruct((NC * g_pad,), F32)),
        mesh=_mesh(),
        compiler_params=_SC_PARAMS,
        scratch_types=[
            pltpu.VMEM((ch,), F32), pltpu.VMEM((ch,), F32),
            pltpu.VMEM((ch,), F32), pltpu.VMEM((ch,), F32),
            pltpu.VMEM((ch,), I32),
            pltpu.VMEM((g_pad,), F32), pltpu.VMEM((g_pad,), F32),
            pltpu.VMEM((g_sl,), F32), pltpu.VMEM((g_sl,), F32),
            pltpu.VMEM((g_sl,), F32),
            pltpu.VMEM_SHARED((NS * g_pad,), F32),
            pltpu.VMEM_SHARED((NS * g_pad,), F32),
        ],
    )
    def k(tp_hbm, u_hbm, dis_hbm, bat_hbm, pool_out, cnt_out, t0_v, t1_v,
          u_v, dis_v, bat_v, locp_v, locc_v, sump_v, sumc_v, tmp_v,
          stage_p, stage_c):
        cid = lax.axis_index("c")
        sid = lax.axis_index("s")
        wid = cid * NS + sid
        base = wid * ch
        _zero_fill(locp_v, g_pad)
        _zero_fill(locc_v, g_pad)
        pltpu.sync_copy(tp_hbm.at[pl.ds(base, ch)], t0_v)
        pltpu.sync_copy(tp_hbm.at[pl.ds(n_pad + base, ch)], t1_v)
        pltpu.sync_copy(u_hbm.at[pl.ds(base, ch)], u_v)
        pltpu.sync_copy(dis_hbm.at[pl.ds(base, ch)], dis_v)
        pltpu.sync_copy(bat_hbm.at[pl.ds(base, ch)], bat_v)
        ov = jnp.ones((L,), F32)

        def zbody(i, _):
            s = pl.ds(i * L, L)
            z = dis_v[s] * (t0_v[s] + t1_v[s] + u_v[s])
            idx = bat_v[s]
            plsc.addupdate_scatter(locp_v, [idx], z)
            plsc.addupdate_scatter(locc_v, [idx], ov)
            return 0

        lax.fori_loop(0, ch // L, zbody, 0)
        pltpu.sync_copy(locp_v, stage_p.at[pl.ds(sid * g_pad, g_pad)])
        pltpu.sync_copy(locc_v, stage_c.at[pl.ds(sid * g_pad, g_pad)])
        plsc.subcore_barrier()
        _zero_fill(sump_v, g_sl)
        _zero_fill(sumc_v, g_sl)
        for j in range(NS):
            pltpu.sync_copy(stage_p.at[pl.ds(j * g_pad + sid * g_sl, g_sl)],
                            tmp_v)
            for i in range(g_sl // L):
                s = pl.ds(i * L, L)
                sump_v[s] = sump_v[s] + tmp_v[s]
            pltpu.sync_copy(stage_c.at[pl.ds(j * g_pad + sid * g_sl, g_sl)],
                            tmp_v)
            for i in range(g_sl // L):
                s = pl.ds(i * L, L)
                sumc_v[s] = sumc_v[s] + tmp_v[s]
        pltpu.sync_copy(sump_v, pool_out.at[pl.ds(cid * g_pad + sid * g_sl,
                                                  g_sl)])
        pltpu.sync_copy(sumc_v, cnt_out.at[pl.ds(cid * g_pad + sid * g_sl,
                                                 g_sl)])

    return k(t_p, u, dis, batch_pad)


def _tc_final(pool_p, cnt_p, b1, W2, b2, g_pad):
    gr = g_pad // 128

    def body(pool_ref, cnt_ref, b1_ref, w2_ref, b2_ref, o_ref):
        c1s = jnp.sum(b1_ref[...] * w2_ref[...])
        pooled = pool_ref[0] + pool_ref[1]
        cnt = cnt_ref[0] + cnt_ref[1]
        o_ref[...] = pooled + cnt * c1s + b2_ref[0, 0]

    out = pl.pallas_call(
        body,
        out_shape=jax.ShapeDtypeStruct((gr, 128), F32),
    )(pool_p.reshape(NC, gr, 128), cnt_p.reshape(NC, gr, 128),
      b1.reshape(1, -1), W2.reshape(1, -1), b2.reshape(1, 1))
    return out.reshape(g_pad)


def kernel(x, edge_index, batch, W1, b1, W2, b2):
    n = x.shape[0]
    e = edge_index.shape[1]
    g = 512
    g_pad = 1024
    n_pad = ((n + 1 + ALIGN - 1) // ALIGN) * ALIGN
    e_pad = ((e + ALIGN - 1) // ALIGN) * ALIGN

    src_flat = jnp.concatenate([edge_index[0], jnp.zeros((e_pad - e,), I32)])
    dst_flat = jnp.concatenate([edge_index[1], jnp.full((e_pad - e,), n, I32)])
    batch_pad = jnp.concatenate([batch, jnp.full((n_pad - n,), g, I32)])

    deg_p = _sc_deg(dst_flat, n_pad)
    y = _tc_matvec(x, W1, W2)
    y_pad = jnp.concatenate([y[:, 0], jnp.zeros((n_pad - n,), F32)])
    dis, u0 = _tc_dis_u0(deg_p, y_pad)
    t1_p = _sc_hop(src_flat, dst_flat, u0)
    u1 = _tc_u_next(t1_p, u0, dis)
    t2_p = _sc_hop(src_flat, dst_flat, u1)
    pool_p, cnt_p = _sc_pool(t2_p, u1, dis, batch_pad, g_pad)
    out = _tc_final(pool_p, cnt_p, b1, W2, b2, g_pad)
    return out[:g].reshape(g, 1)
